# baseline (speedup 1.0000x reference)
import functools

import jax
import jax.numpy as jnp
from jax import lax
from jax.experimental import pallas as pl
from jax.experimental.pallas import tpu as pltpu
from jax.experimental.pallas import tpu_sc as plsc

IN_DIM = 1000000
DEPTH = 64
SEQ = 200
BATCH = 4096
ROWS = BATCH * SEQ
NC = 2
NS = 16
LANES = 16
NW = NC * NS
PER_W = ROWS // NW
CHUNK = 400
NCHUNK = PER_W // CHUNK
GSZ = 100
NG = CHUNK // GSZ
TOTAL_CHUNKS = ROWS // CHUNK


def _pos_encoding():
    half = DEPTH // 2
    positions = jnp.arange(SEQ, dtype=jnp.float32)[:, None]
    depths = jnp.arange(half, dtype=jnp.float32)[None, :] / half
    angle_rates = 1.0 / 10000.0 ** depths
    angle_rads = positions * angle_rates
    return jnp.concatenate([jnp.sin(angle_rads), jnp.cos(angle_rads)], axis=-1)


def _make_sc_kernel():
    mesh = plsc.VectorSubcoreMesh(core_axis_name="c", subcore_axis_name="s")

    @functools.partial(
        pl.kernel,
        mesh=mesh,
        compiler_params=pltpu.CompilerParams(use_tc_tiling_on_sc=False),
        out_type=jax.ShapeDtypeStruct((BATCH, SEQ, DEPTH), jnp.float32),
        scratch_types=[
            pltpu.VMEM((2, NG, GSZ), jnp.int32),
            pltpu.VMEM((2, CHUNK, DEPTH), jnp.float32),
            pltpu.VMEM((CHUNK, DEPTH), jnp.float32),
            pltpu.SemaphoreType.DMA,
            pltpu.SemaphoreType.DMA,
            pltpu.SemaphoreType.DMA,
            pltpu.SemaphoreType.DMA,
        ],
    )
    def k(idx_hbm, table_hbm, pos_hbm, out_hbm, idx_v, rows_v, pos_v,
          gsem0, gsem1, wsem0, wsem1):
        wid = lax.axis_index("s") * NC + lax.axis_index("c")
        c0 = wid * NCHUNK
        gsems = (gsem0, gsem1)
        wsems = (wsem0, wsem1)

        pltpu.sync_copy(pos_hbm, pos_v)

        def fire_chunk(c, buf):
            pltpu.sync_copy(idx_hbm.at[c0 + c], idx_v.at[buf])
            for g in range(NG):
                pltpu.async_copy(
                    table_hbm.at[idx_v.at[buf, g]],
                    rows_v.at[buf, pl.ds(g * GSZ, GSZ)],
                    gsems[buf],
                )

        def wait_gathers(buf):
            for g in range(NG):
                pltpu.make_async_copy(
                    table_hbm.at[idx_v.at[buf, 0]],
                    rows_v.at[buf, pl.ds(g * GSZ, GSZ)],
                    gsems[buf],
                ).wait()

        def fire_writes(c, buf):
            for h in range(2):
                pltpu.async_copy(
                    rows_v.at[buf, pl.ds(h * SEQ, SEQ)],
                    out_hbm.at[2 * (c0 + c) + h],
                    wsems[buf],
                )

        def wait_writes(c, buf):
            for h in range(2):
                pltpu.make_async_copy(
                    rows_v.at[buf, pl.ds(h * SEQ, SEQ)],
                    out_hbm.at[2 * (c0 + c) + h],
                    wsems[buf],
                ).wait()

        fire_chunk(0, 0)

        def iter_body(i, carry):
            for buf in range(2):
                c = 2 * i + buf

                @pl.when(jnp.logical_and(c + 1 < NCHUNK, c >= 1))
                def _():
                    wait_writes(c - 1, 1 - buf)

                @pl.when(c + 1 < NCHUNK)
                def _():
                    fire_chunk(c + 1, 1 - buf)

                wait_gathers(buf)

                def row_body(r, rcarry, buf=buf):
                    for j in range(DEPTH // LANES):
                        sl = pl.ds(j * LANES, LANES)
                        plsc.addupdate(rows_v.at[buf, r, sl], pos_v[r, sl])
                    return rcarry

                lax.fori_loop(0, CHUNK, row_body, 0)
                fire_writes(c, buf)
            return carry

        lax.fori_loop(0, NCHUNK // 2, iter_body, 0)
        wait_writes(NCHUNK - 2, 0)
        wait_writes(NCHUNK - 1, 1)

    return k


def kernel(seq, table):
    idx = seq.astype(jnp.int32).reshape(TOTAL_CHUNKS, NG, GSZ)
    pos_tiled = jnp.tile(_pos_encoding(), (CHUNK // SEQ, 1))
    return _make_sc_kernel()(idx, table, pos_tiled)
